# baseline (speedup 1.0000x reference)
import functools

import numpy as np
import jax
import jax.numpy as jnp
from jax import lax
from jax.experimental import pallas as pl
from jax.experimental.pallas import tpu as pltpu
from jax.experimental.pallas import tpu_sc as plsc

EMBED = 32
MAX_LEN = 202
SEQ = 200
BATCH = 4096
N_TOK = BATCH * SEQ
NUM_WORKERS = 32
PER_W = N_TOK // NUM_WORKERS
CHUNK = 2 * SEQ
N_CHUNKS = PER_W // CHUNK
N_EH = 39
N_COMBO = 52 * N_EH
UNROLL = 8


def _make_pe() -> jnp.ndarray:
    pos = np.arange(MAX_LEN, dtype=np.float32)[:, None]
    div = np.exp(np.arange(0, EMBED, 2, dtype=np.float32) * -(np.log(10000.0) / EMBED))
    pe = np.zeros((MAX_LEN, EMBED), dtype=np.float32)
    pe[:, 0::2] = np.sin(pos * div)
    pe[:, 1::2] = np.cos(pos * div)
    return jnp.asarray(pe[:SEQ])


_MESH = plsc.VectorSubcoreMesh(core_axis_name="c", subcore_axis_name="s")


@functools.partial(
    pl.kernel,
    out_type=jax.ShapeDtypeStruct((N_TOK, EMBED), jnp.float32),
    mesh=_MESH,
    compiler_params=pltpu.CompilerParams(use_tc_tiling_on_sc=False,
                                         needs_layout_passes=False),
    scratch_types=[
        pltpu.VMEM((3, CHUNK), jnp.int32),
        pltpu.VMEM((3, CHUNK), jnp.int32),
        pltpu.VMEM((3, CHUNK, EMBED), jnp.float32),
        pltpu.VMEM((52, EMBED), jnp.float32),
        pltpu.VMEM((13, EMBED), jnp.float32),
        pltpu.VMEM((3, EMBED), jnp.float32),
        pltpu.VMEM((N_EH, EMBED), jnp.float32),
        pltpu.VMEM((N_COMBO, EMBED), jnp.float32),
        pltpu.VMEM((CHUNK, EMBED), jnp.float32),
        pltpu.SemaphoreType.DMA,
        pltpu.SemaphoreType.DMA,
        pltpu.SemaphoreType.DMA,
        pltpu.SemaphoreType.DMA,
        pltpu.SemaphoreType.DMA,
        pltpu.SemaphoreType.DMA,
        pltpu.SemaphoreType.DMA,
        pltpu.SemaphoreType.DMA,
        pltpu.SemaphoreType.DMA,
    ],
)
def _emb_kernel(grid_tab, time_tab, event_tab, hand_tab, pe_tab, gidx, cidx, out,
                s_gi, s_ci, acc, time_v, event_v, hand_v, eh_v, combo_v, pe2_v,
                sem_i0, sem_i1, sem_i2, sem_g0, sem_g1, sem_g2,
                sem_w0, sem_w1, sem_w2):
    sem_i = (sem_i0, sem_i1, sem_i2)
    sem_g = (sem_g0, sem_g1, sem_g2)
    sem_w = (sem_w0, sem_w1, sem_w2)
    wid = lax.axis_index("s") * 2 + lax.axis_index("c")
    tok0 = wid * PER_W

    pltpu.sync_copy(time_tab, time_v)
    pltpu.sync_copy(event_tab, event_v)
    pltpu.sync_copy(hand_tab, hand_v)
    pltpu.sync_copy(pe_tab, pe2_v.at[pl.ds(0, SEQ)])
    pltpu.sync_copy(pe_tab, pe2_v.at[pl.ds(SEQ, SEQ)])

    def eh_body(j, carry):
        e = j // 3
        h = j - e * 3
        for c0 in (0, 16):
            eh_v[j, c0:c0 + 16] = event_v[e, c0:c0 + 16] + hand_v[h, c0:c0 + 16]
        return carry

    lax.fori_loop(0, N_EH, eh_body, 0)

    def combo_body(r, carry):
        t = r // N_EH
        j = r - t * N_EH
        for c0 in (0, 16):
            combo_v[r, c0:c0 + 16] = time_v[t, c0:c0 + 16] + eh_v[j, c0:c0 + 16]
        return carry

    lax.fori_loop(0, N_COMBO, combo_body, 0)

    iota16 = lax.iota(jnp.int32, 16)
    col_vecs = [lax.bitwise_and(iota16 + d, EMBED - 1) for d in range(EMBED)]

    def fire_idx(i, slot):
        base = tok0 + i * CHUNK
        pltpu.async_copy(gidx.at[pl.ds(base, CHUNK)], s_gi.at[slot], sem_i[slot])
        pltpu.async_copy(cidx.at[pl.ds(base, CHUNK)], s_ci.at[slot], sem_i[slot])

    def wait_idx(slot):
        pltpu.make_async_copy(gidx.at[pl.ds(0, CHUNK)], s_gi.at[slot], sem_i[slot]).wait()
        pltpu.make_async_copy(cidx.at[pl.ds(0, CHUNK)], s_ci.at[slot], sem_i[slot]).wait()

    def fire_gather(slot):
        pltpu.async_copy(grid_tab.at[s_gi.at[slot]], acc.at[slot], sem_g[slot])

    def wait_gather(slot):
        pltpu.make_async_copy(grid_tab.at[s_gi.at[slot]], acc.at[slot], sem_g[slot]).wait()

    def fire_wb(i, slot):
        base = tok0 + i * CHUNK
        pltpu.async_copy(acc.at[slot], out.at[pl.ds(base, CHUNK)], sem_w[slot])

    def wait_wb(slot):
        pltpu.make_async_copy(acc.at[slot], out.at[pl.ds(0, CHUNK)], sem_w[slot]).wait()

    def compute(slot):
        def tok_body(g, c2):
            row16 = iota16 + g * 16
            c16 = s_ci[slot, pl.ds(g * 16, 16)]
            for col16 in col_vecs:
                v = (plsc.load_gather(combo_v, [c16, col16])
                     + plsc.load_gather(pe2_v, [row16, col16]))
                plsc.addupdate_scatter(acc.at[slot], [row16, col16], v)
            return c2

        lax.fori_loop(0, CHUNK // 16, tok_body, 0)

    def step(i, s):
        s2 = (s + 2) % 3

        @pl.when(i + 2 < N_CHUNKS)
        def _():
            @pl.when(i >= 1)
            def _():
                wait_wb(s2)

            wait_idx(s2)
            fire_gather(s2)

        wait_gather(s)
        compute(s)
        fire_wb(i, s)

        @pl.when(i + 3 < N_CHUNKS)
        def _():
            fire_idx(i + 3, s)

    fire_idx(0, 0)
    fire_idx(1, 1)
    fire_idx(2, 2)
    wait_idx(0)
    fire_gather(0)
    wait_idx(1)
    fire_gather(1)

    def triple_body(pr, carry):
        step(3 * pr, 0)
        step(3 * pr + 1, 1)
        step(3 * pr + 2, 2)
        return carry

    lax.fori_loop(0, (N_CHUNKS - 1) // 3, triple_body, 0)
    step(N_CHUNKS - 1, (N_CHUNKS - 1) % 3)
    wait_wb((N_CHUNKS - 3) % 3)
    wait_wb((N_CHUNKS - 2) % 3)
    wait_wb((N_CHUNKS - 1) % 3)


def kernel(grid, timestamp, event, hand, grid_table, time_table, event_table,
           hand_table, train_mode):
    pe = _make_pe()
    gi = grid.astype(jnp.int32).reshape(N_TOK)
    ci = (timestamp.astype(jnp.int32) * N_EH + event.astype(jnp.int32) * 3
          + hand.astype(jnp.int32)).reshape(N_TOK)
    out = _emb_kernel(grid_table, time_table, event_table, hand_table, pe, gi, ci)
    return out.reshape(BATCH, SEQ, EMBED)


# speedup vs baseline: 1.0117x; 1.0117x over previous
import functools

import numpy as np
import jax
import jax.numpy as jnp
from jax import lax
from jax.experimental import pallas as pl
from jax.experimental.pallas import tpu as pltpu
from jax.experimental.pallas import tpu_sc as plsc

EMBED = 32
MAX_LEN = 202
SEQ = 200
BATCH = 4096
N_TOK = BATCH * SEQ
NUM_WORKERS = 32
PER_W = N_TOK // NUM_WORKERS
CHUNK = 2 * SEQ
N_CHUNKS = PER_W // CHUNK
N_EH = 39
N_COMBO = 52 * N_EH
UNROLL = 8


def _make_pe() -> jnp.ndarray:
    pos = np.arange(MAX_LEN, dtype=np.float32)[:, None]
    div = np.exp(np.arange(0, EMBED, 2, dtype=np.float32) * -(np.log(10000.0) / EMBED))
    pe = np.zeros((MAX_LEN, EMBED), dtype=np.float32)
    pe[:, 0::2] = np.sin(pos * div)
    pe[:, 1::2] = np.cos(pos * div)
    return jnp.asarray(pe[:SEQ])


_MESH = plsc.VectorSubcoreMesh(core_axis_name="c", subcore_axis_name="s")


@functools.partial(
    pl.kernel,
    out_type=jax.ShapeDtypeStruct((N_TOK, EMBED), jnp.float32),
    mesh=_MESH,
    compiler_params=pltpu.CompilerParams(use_tc_tiling_on_sc=False,
                                         needs_layout_passes=False),
    scratch_types=[
        pltpu.VMEM((3, CHUNK), jnp.int32),
        pltpu.VMEM((3, CHUNK), jnp.int32),
        pltpu.VMEM((3, CHUNK, EMBED), jnp.float32),
        pltpu.VMEM((52, EMBED), jnp.float32),
        pltpu.VMEM((13, EMBED), jnp.float32),
        pltpu.VMEM((3, EMBED), jnp.float32),
        pltpu.VMEM((N_EH, EMBED), jnp.float32),
        pltpu.VMEM((N_COMBO, EMBED), jnp.float32),
        pltpu.VMEM((CHUNK, EMBED), jnp.float32),
        pltpu.SemaphoreType.DMA,
        pltpu.SemaphoreType.DMA,
        pltpu.SemaphoreType.DMA,
        pltpu.SemaphoreType.DMA,
        pltpu.SemaphoreType.DMA,
        pltpu.SemaphoreType.DMA,
        pltpu.SemaphoreType.DMA,
        pltpu.SemaphoreType.DMA,
        pltpu.SemaphoreType.DMA,
    ],
)
def _emb_kernel(grid_tab, time_tab, event_tab, hand_tab, pe_tab, gidx, cidx, out,
                s_gi, s_ci, acc, time_v, event_v, hand_v, eh_v, combo_v, pe2_v,
                sem_i0, sem_i1, sem_i2, sem_g0, sem_g1, sem_g2,
                sem_w0, sem_w1, sem_w2):
    sem_i = (sem_i0, sem_i1, sem_i2)
    sem_g = (sem_g0, sem_g1, sem_g2)
    sem_w = (sem_w0, sem_w1, sem_w2)
    wid = lax.axis_index("s") * 2 + lax.axis_index("c")
    tok0 = wid * PER_W

    pltpu.sync_copy(time_tab, time_v)
    pltpu.sync_copy(event_tab, event_v)
    pltpu.sync_copy(hand_tab, hand_v)
    pltpu.sync_copy(pe_tab, pe2_v.at[pl.ds(0, SEQ)])
    pltpu.sync_copy(pe_tab, pe2_v.at[pl.ds(SEQ, SEQ)])

    def eh_body(j, carry):
        e = j // 3
        h = j - e * 3
        for c0 in (0, 16):
            eh_v[j, c0:c0 + 16] = event_v[e, c0:c0 + 16] + hand_v[h, c0:c0 + 16]
        return carry

    lax.fori_loop(0, N_EH, eh_body, 0)

    def combo_body(r, carry):
        t = r // N_EH
        j = r - t * N_EH
        for c0 in (0, 16):
            combo_v[r, c0:c0 + 16] = time_v[t, c0:c0 + 16] + eh_v[j, c0:c0 + 16]
        return carry

    lax.fori_loop(0, N_COMBO, combo_body, 0)

    iota16 = lax.iota(jnp.int32, 16)
    col_vecs = [lax.bitwise_and(iota16 + d, EMBED - 1) for d in range(EMBED)]

    def fire_idx(i, slot):
        base = tok0 + i * CHUNK
        pltpu.async_copy(gidx.at[pl.ds(base, CHUNK)], s_gi.at[slot], sem_i[slot])
        pltpu.async_copy(cidx.at[pl.ds(base, CHUNK)], s_ci.at[slot], sem_i[slot])

    def wait_idx(slot):
        pltpu.make_async_copy(gidx.at[pl.ds(0, CHUNK)], s_gi.at[slot], sem_i[slot]).wait()
        pltpu.make_async_copy(cidx.at[pl.ds(0, CHUNK)], s_ci.at[slot], sem_i[slot]).wait()

    def fire_gather(slot):
        pltpu.async_copy(grid_tab.at[s_gi.at[slot]], acc.at[slot], sem_g[slot])

    def wait_gather(slot):
        pltpu.make_async_copy(grid_tab.at[s_gi.at[slot]], acc.at[slot], sem_g[slot]).wait()

    def fire_wb(i, slot):
        base = tok0 + i * CHUNK
        pltpu.async_copy(acc.at[slot], out.at[pl.ds(base, CHUNK)], sem_w[slot])

    def wait_wb(slot):
        pltpu.make_async_copy(acc.at[slot], out.at[pl.ds(0, CHUNK)], sem_w[slot]).wait()

    def compute(slot):
        GU = 5

        def tok_body(gg, c2):
            for u in range(GU):
                g = gg * GU + u
                row16 = iota16 + g * 16
                c16 = s_ci[slot, pl.ds(g * 16, 16)]
                for col16 in col_vecs:
                    v = (plsc.load_gather(combo_v, [c16, col16])
                         + plsc.load_gather(pe2_v, [row16, col16]))
                    plsc.addupdate_scatter(acc.at[slot], [row16, col16], v)
            return c2

        lax.fori_loop(0, CHUNK // 16 // GU, tok_body, 0)

    def step(i, s):
        s2 = (s + 2) % 3

        @pl.when(i + 2 < N_CHUNKS)
        def _():
            @pl.when(i >= 1)
            def _():
                wait_wb(s2)

            wait_idx(s2)
            fire_gather(s2)

        wait_gather(s)
        compute(s)
        fire_wb(i, s)

        @pl.when(i + 3 < N_CHUNKS)
        def _():
            fire_idx(i + 3, s)

    fire_idx(0, 0)
    fire_idx(1, 1)
    fire_idx(2, 2)
    wait_idx(0)
    fire_gather(0)
    wait_idx(1)
    fire_gather(1)

    def triple_body(pr, carry):
        step(3 * pr, 0)
        step(3 * pr + 1, 1)
        step(3 * pr + 2, 2)
        return carry

    lax.fori_loop(0, (N_CHUNKS - 1) // 3, triple_body, 0)
    step(N_CHUNKS - 1, (N_CHUNKS - 1) % 3)
    wait_wb((N_CHUNKS - 3) % 3)
    wait_wb((N_CHUNKS - 2) % 3)
    wait_wb((N_CHUNKS - 1) % 3)


def kernel(grid, timestamp, event, hand, grid_table, time_table, event_table,
           hand_table, train_mode):
    pe = _make_pe()
    gi = grid.astype(jnp.int32).reshape(N_TOK)
    ci = (timestamp.astype(jnp.int32) * N_EH + event.astype(jnp.int32) * 3
          + hand.astype(jnp.int32)).reshape(N_TOK)
    out = _emb_kernel(grid_table, time_table, event_table, hand_table, pe, gi, ci)
    return out.reshape(BATCH, SEQ, EMBED)
